# Initial kernel scaffold; baseline (speedup 1.0000x reference)
import jax
import jax.numpy as jnp
from jax.experimental import pallas as pl


def kernel(x, edge_index, edge_attr, atom_tables, bond_tables, W1, b1, W2, b2, eps, g1, be1, g2, be2):
    raise NotImplementedError("write your pallas kernel here")


# speedup vs baseline: 2.9421x; 2.9421x over previous
import functools

import jax
import jax.numpy as jnp
from jax import lax
from jax.experimental import pallas as pl
from jax.experimental.pallas import tpu as pltpu
from jax.experimental.pallas import tpu_sc as plsc

N = 10000
E = 320000
D = 128
L = 3
ATOM_F = 9
ATOM_V = 101

NC = 2
NS = 16
NW = NC * NS

NP = 10240
ZROWS = NP // NS
DROWS = 640

E_ICOPY = 10
E_ROWS_W = E_ICOPY * 8
E_PAD = NW * E_ROWS_W * 128

A_ICOPY = 6
A_ROWS_W = A_ICOPY * 8
P = N * ATOM_F
P_PAD = NS * A_ROWS_W * 128

_MESH = plsc.VectorSubcoreMesh(core_axis_name="c", subcore_axis_name="s")


def _atom_body(tab_hbm, idx_hbm, nid_hbm, zrow_hbm, out_hbm,
               idx_v, nid_v, rows_v, agg_sp, sem):
    cid = lax.axis_index("c")
    sid = lax.axis_index("s")

    @pl.when(cid == 0)
    def _():
        zoff = pl.multiple_of(sid * ZROWS, 8)
        pltpu.sync_copy(zrow_hbm, agg_sp.at[pl.ds(zoff, ZROWS)])
        plsc.subcore_barrier()
        for i in range(A_ICOPY):
            r0 = pl.multiple_of(sid * A_ROWS_W + i * 8, 8)
            pltpu.sync_copy(idx_hbm.at[pl.ds(r0, 8)], idx_v)
            pltpu.sync_copy(nid_hbm.at[pl.ds(r0, 8)], nid_v)
            for j in range(8):
                pltpu.async_copy(tab_hbm.at[idx_v.at[j]], rows_v, sem).wait()
                pltpu.sync_copy(rows_v, agg_sp.at[nid_v.at[j]], add=True)
        plsc.subcore_barrier()

        @pl.when(sid < NS - 1)
        def _():
            doff = pl.multiple_of(sid * DROWS, 8)
            pltpu.sync_copy(agg_sp.at[pl.ds(doff, DROWS)],
                            out_hbm.at[pl.ds(doff, DROWS)])

        @pl.when(sid == NS - 1)
        def _():
            pltpu.sync_copy(agg_sp.at[pl.ds((NS - 1) * DROWS, N - (NS - 1) * DROWS)],
                            out_hbm.at[pl.ds((NS - 1) * DROWS, N - (NS - 1) * DROWS)])


_atom_encode = pl.kernel(
    _atom_body,
    out_type=jax.ShapeDtypeStruct((N, D), jnp.float32),
    mesh=_MESH,
    scratch_types=[
        pltpu.VMEM((8, 128), jnp.int32),
        pltpu.VMEM((8, 128), jnp.int32),
        pltpu.VMEM((128, D), jnp.float32),
        pltpu.VMEM_SHARED((NP, D), jnp.float32),
        pltpu.SemaphoreType.DMA,
    ],
)


def _layer_body(h_hbm, src_hbm, dst_hbm, code_hbm, bond_hbm, zrow_hbm, out_hbm,
                src_v, dst_v, code_v, hrows, eerows, agg_sp, sem):
    cid = lax.axis_index("c")
    sid = lax.axis_index("s")
    w = cid * NS + sid

    zoff = pl.multiple_of(sid * ZROWS, 8)
    pltpu.sync_copy(zrow_hbm, agg_sp.at[pl.ds(zoff, ZROWS)])
    plsc.subcore_barrier()

    for i in range(E_ICOPY):
        r0 = pl.multiple_of(w * E_ROWS_W + i * 8, 8)
        pltpu.sync_copy(src_hbm.at[pl.ds(r0, 8)], src_v)
        pltpu.sync_copy(dst_hbm.at[pl.ds(r0, 8)], dst_v)
        pltpu.sync_copy(code_hbm.at[pl.ds(r0, 8)], code_v)
        for q in range(8):
            d0 = pltpu.async_copy(h_hbm.at[src_v.at[q]], hrows, sem)
            d1 = pltpu.async_copy(bond_hbm.at[code_v.at[q]], eerows, sem)
            d0.wait()
            d1.wait()

            def vbody(r, _):
                for c in range(D // 16):
                    sl = pl.ds(c * 16, 16)
                    hrows[r, sl] = jnp.maximum(hrows[r, sl] + eerows[r, sl], 0.0)
                return 0

            lax.fori_loop(0, 128, vbody, 0)

            pltpu.sync_copy(hrows, agg_sp.at[dst_v.at[q]], add=True)

    plsc.subcore_barrier()

    @pl.when(sid < NS - 1)
    def _():
        doff = pl.multiple_of(sid * DROWS, 8)
        pltpu.sync_copy(agg_sp.at[pl.ds(doff, DROWS)],
                        out_hbm.at[cid, pl.ds(doff, DROWS)])

    @pl.when(sid == NS - 1)
    def _():
        pltpu.sync_copy(agg_sp.at[pl.ds((NS - 1) * DROWS, N - (NS - 1) * DROWS)],
                        out_hbm.at[cid, pl.ds((NS - 1) * DROWS, N - (NS - 1) * DROWS)])


_layer_agg = pl.kernel(
    _layer_body,
    out_type=jax.ShapeDtypeStruct((NC, N, D), jnp.float32),
    mesh=_MESH,
    scratch_types=[
        pltpu.VMEM((8, 128), jnp.int32),
        pltpu.VMEM((8, 128), jnp.int32),
        pltpu.VMEM((8, 128), jnp.int32),
        pltpu.VMEM((128, D), jnp.float32),
        pltpu.VMEM((128, D), jnp.float32),
        pltpu.VMEM_SHARED((NP, D), jnp.float32),
        pltpu.SemaphoreType.DMA,
    ],
)


def _mlp_body(relu_last, h_ref, p_ref, s_ref, w1_ref, b1_ref, g1_ref, be1_ref,
              w2_ref, b2_ref, g2_ref, be2_ref, out_ref):
    t = s_ref[0, 0] * h_ref[...] + p_ref[0] + p_ref[1]
    u = jnp.dot(t, w1_ref[...], preferred_element_type=jnp.float32) + b1_ref[...]
    mu = jnp.mean(u, axis=0, keepdims=True)
    var = jnp.mean((u - mu) * (u - mu), axis=0, keepdims=True)
    u = g1_ref[...] * (u - mu) * lax.rsqrt(var + 1e-5) + be1_ref[...]
    u = jnp.maximum(u, 0.0)
    v = jnp.dot(u, w2_ref[...], preferred_element_type=jnp.float32) + b2_ref[...]
    mu2 = jnp.mean(v, axis=0, keepdims=True)
    var2 = jnp.mean((v - mu2) * (v - mu2), axis=0, keepdims=True)
    v = g2_ref[...] * (v - mu2) * lax.rsqrt(var2 + 1e-5) + be2_ref[...]
    if relu_last:
        v = jnp.maximum(v, 0.0)
    out_ref[...] = v


def _mlp(h, parts, s, w1, b1, g1, be1, w2, b2, g2, be2, relu_last):
    return pl.pallas_call(
        functools.partial(_mlp_body, relu_last),
        out_shape=jax.ShapeDtypeStruct((N, D), jnp.float32),
    )(h, parts, s, w1, b1, g1, be1, w2, b2, g2, be2)


def kernel(x, edge_index, edge_attr, atom_tables, bond_tables,
           W1, b1, W2, b2, eps, g1, be1, g2, be2):
    f32 = jnp.float32
    zrow = jnp.zeros((ZROWS, D), f32)

    tabflat = atom_tables.reshape(ATOM_F * ATOM_V, D).astype(f32)
    xflat = (x.astype(jnp.int32) + jnp.arange(ATOM_F, dtype=jnp.int32)[None, :] * ATOM_V
             ).reshape(-1)
    nid = jnp.repeat(jnp.arange(N, dtype=jnp.int32), ATOM_F)
    xflat = jnp.concatenate([xflat, jnp.zeros((P_PAD - P,), jnp.int32)]
                            ).reshape(P_PAD // 128, 128)
    nid = jnp.concatenate([nid, jnp.full((P_PAD - P,), N, jnp.int32)]
                          ).reshape(P_PAD // 128, 128)

    h = _atom_encode(tabflat, xflat, nid, zrow)

    src = edge_index[0].astype(jnp.int32)
    dst = edge_index[1].astype(jnp.int32)
    ea = edge_attr.astype(jnp.int32)
    code = ea[:, 0] * 25 + ea[:, 1] * 5 + ea[:, 2]
    pad = E_PAD - E
    src = jnp.concatenate([src, jnp.zeros((pad,), jnp.int32)]).reshape(E_PAD // 128, 128)
    dst = jnp.concatenate([dst, jnp.full((pad,), N, jnp.int32)]).reshape(E_PAD // 128, 128)
    code = jnp.concatenate([code, jnp.zeros((pad,), jnp.int32)]).reshape(E_PAD // 128, 128)

    bond = (bond_tables[:, 0][:, :, None, None, :]
            + bond_tables[:, 1][:, None, :, None, :]
            + bond_tables[:, 2][:, None, None, :, :]).reshape(L, 125, D)
    bond = jnp.concatenate([bond, jnp.zeros((L, 3, D), f32)], axis=1)

    for l in range(L):
        parts = _layer_agg(h, src, dst, code, bond[l], zrow)
        s = jnp.reshape(1.0 + eps[l], (1, 1)).astype(f32)
        h = _mlp(h, parts, s,
                 W1[l], b1[l].reshape(1, 2 * D), g1[l].reshape(1, 2 * D),
                 be1[l].reshape(1, 2 * D),
                 W2[l], b2[l].reshape(1, D), g2[l].reshape(1, D),
                 be2[l].reshape(1, D), relu_last=(l < L - 1))
    return h


# speedup vs baseline: 3.3867x; 1.1511x over previous
import functools

import jax
import jax.numpy as jnp
from jax import lax
from jax.experimental import pallas as pl
from jax.experimental.pallas import tpu as pltpu
from jax.experimental.pallas import tpu_sc as plsc

N = 10000
E = 320000
D = 128
L = 3
ATOM_F = 9
ATOM_V = 101

NC = 2
NS = 16
NW = NC * NS

NP = 10240
ZROWS = NP // NS
DROWS = 640

E_ICOPY = 10
E_ROWS_W = E_ICOPY * 8
E_PAD = NW * E_ROWS_W * 128

A_ICOPY = 6
A_ROWS_W = A_ICOPY * 8
P = N * ATOM_F
P_PAD = NS * A_ROWS_W * 128

_MESH = plsc.VectorSubcoreMesh(core_axis_name="c", subcore_axis_name="s")


def _atom_body(tab_hbm, idx_hbm, nid_hbm, zrow_hbm, out_hbm,
               idx_v, nid_v, rows_v, agg_sp, sem):
    cid = lax.axis_index("c")
    sid = lax.axis_index("s")

    @pl.when(cid == 0)
    def _():
        zoff = pl.multiple_of(sid * ZROWS, 8)
        pltpu.sync_copy(zrow_hbm, agg_sp.at[pl.ds(zoff, ZROWS)])
        plsc.subcore_barrier()
        for i in range(A_ICOPY):
            r0 = pl.multiple_of(sid * A_ROWS_W + i * 8, 8)
            pltpu.sync_copy(idx_hbm.at[pl.ds(r0, 8)], idx_v)
            pltpu.sync_copy(nid_hbm.at[pl.ds(r0, 8)], nid_v)
            for j in range(8):
                pltpu.async_copy(tab_hbm.at[idx_v.at[j]], rows_v, sem).wait()
                pltpu.sync_copy(rows_v, agg_sp.at[nid_v.at[j]], add=True)
        plsc.subcore_barrier()

        @pl.when(sid < NS - 1)
        def _():
            doff = pl.multiple_of(sid * DROWS, 8)
            pltpu.sync_copy(agg_sp.at[pl.ds(doff, DROWS)],
                            out_hbm.at[pl.ds(doff, DROWS)])

        @pl.when(sid == NS - 1)
        def _():
            pltpu.sync_copy(agg_sp.at[pl.ds((NS - 1) * DROWS, N - (NS - 1) * DROWS)],
                            out_hbm.at[pl.ds((NS - 1) * DROWS, N - (NS - 1) * DROWS)])


_atom_encode = pl.kernel(
    _atom_body,
    out_type=jax.ShapeDtypeStruct((N, D), jnp.float32),
    mesh=_MESH,
    scratch_types=[
        pltpu.VMEM((8, 128), jnp.int32),
        pltpu.VMEM((8, 128), jnp.int32),
        pltpu.VMEM((128, D), jnp.float32),
        pltpu.VMEM_SHARED((NP, D), jnp.float32),
        pltpu.SemaphoreType.DMA,
    ],
)


def _layer_body(h_hbm, src_hbm, dst_hbm, code_hbm, bond_hbm, zrow_hbm, out_hbm,
                src_v, dst_v, code_v, hr, ee, agg_sp, sem_g, sem_s0, sem_s1):
    cid = lax.axis_index("c")
    sid = lax.axis_index("s")
    w = cid * NS + sid

    zoff = pl.multiple_of(sid * ZROWS, 8)
    pltpu.sync_copy(zrow_hbm, agg_sp.at[pl.ds(zoff, ZROWS)])
    plsc.subcore_barrier()

    def block(i, _):
        r0 = pl.multiple_of(w * E_ROWS_W + i * 8, 8)
        pltpu.sync_copy(src_hbm.at[pl.ds(r0, 8)], src_v)
        pltpu.sync_copy(dst_hbm.at[pl.ds(r0, 8)], dst_v)
        pltpu.sync_copy(code_hbm.at[pl.ds(r0, 8)], code_v)

        def issue_gather(q):
            s = q % 2
            j, half = q // 2, q % 2
            isl = pl.ds(half * 64, 64)
            return (pltpu.async_copy(h_hbm.at[src_v.at[j, isl]], hr.at[s], sem_g),
                    pltpu.async_copy(bond_hbm.at[code_v.at[j, isl]], ee.at[s], sem_g))

        gd = issue_gather(0)
        sc_pend = [None, None]
        for q in range(16):
            s = q % 2
            gd[0].wait()
            gd[1].wait()
            if sc_pend[1 - s] is not None:
                for dsc in sc_pend[1 - s]:
                    dsc.wait()
                sc_pend[1 - s] = None
            if q < 15:
                gd = issue_gather(q + 1)

            def vbody(r, _):
                for c in range(D // 16):
                    sl = pl.ds(c * 16, 16)
                    hr[s, r, sl] = jnp.maximum(hr[s, r, sl] + ee[s, r, sl], 0.0)
                return 0

            lax.fori_loop(0, 64, vbody, 0)

            j, half = q // 2, q % 2
            sem_s = sem_s0 if s == 0 else sem_s1
            scds = []
            for g in range(4):
                dvec = dst_v[j, pl.ds(half * 64 + g * 16, 16)]
                scds.append(pltpu.async_copy(hr.at[s, pl.ds(g * 16, 16)],
                                             agg_sp.at[dvec], sem_s, add=True))
            sc_pend[s] = scds
        for pend in sc_pend:
            if pend is not None:
                for dsc in pend:
                    dsc.wait()
        return 0

    lax.fori_loop(0, E_ICOPY, block, 0)

    plsc.subcore_barrier()

    @pl.when(sid < NS - 1)
    def _():
        doff = pl.multiple_of(sid * DROWS, 8)
        pltpu.sync_copy(agg_sp.at[pl.ds(doff, DROWS)],
                        out_hbm.at[cid, pl.ds(doff, DROWS)])

    @pl.when(sid == NS - 1)
    def _():
        pltpu.sync_copy(agg_sp.at[pl.ds((NS - 1) * DROWS, N - (NS - 1) * DROWS)],
                        out_hbm.at[cid, pl.ds((NS - 1) * DROWS, N - (NS - 1) * DROWS)])


_layer_agg = pl.kernel(
    _layer_body,
    out_type=jax.ShapeDtypeStruct((NC, N, D), jnp.float32),
    mesh=_MESH,
    scratch_types=[
        pltpu.VMEM((8, 128), jnp.int32),
        pltpu.VMEM((8, 128), jnp.int32),
        pltpu.VMEM((8, 128), jnp.int32),
        pltpu.VMEM((2, 64, D), jnp.float32),
        pltpu.VMEM((2, 64, D), jnp.float32),
        pltpu.VMEM_SHARED((NP, D), jnp.float32),
        pltpu.SemaphoreType.DMA,
        pltpu.SemaphoreType.DMA,
        pltpu.SemaphoreType.DMA,
    ],
)


def _mlp_body(relu_last, h_ref, p_ref, s_ref, w1_ref, b1_ref, g1_ref, be1_ref,
              w2_ref, b2_ref, g2_ref, be2_ref, out_ref):
    t = s_ref[0, 0] * h_ref[...] + p_ref[0] + p_ref[1]
    u = jnp.dot(t, w1_ref[...], preferred_element_type=jnp.float32) + b1_ref[...]
    mu = jnp.mean(u, axis=0, keepdims=True)
    var = jnp.mean((u - mu) * (u - mu), axis=0, keepdims=True)
    u = g1_ref[...] * (u - mu) * lax.rsqrt(var + 1e-5) + be1_ref[...]
    u = jnp.maximum(u, 0.0)
    v = jnp.dot(u, w2_ref[...], preferred_element_type=jnp.float32) + b2_ref[...]
    mu2 = jnp.mean(v, axis=0, keepdims=True)
    var2 = jnp.mean((v - mu2) * (v - mu2), axis=0, keepdims=True)
    v = g2_ref[...] * (v - mu2) * lax.rsqrt(var2 + 1e-5) + be2_ref[...]
    if relu_last:
        v = jnp.maximum(v, 0.0)
    out_ref[...] = v


def _mlp(h, parts, s, w1, b1, g1, be1, w2, b2, g2, be2, relu_last):
    return pl.pallas_call(
        functools.partial(_mlp_body, relu_last),
        out_shape=jax.ShapeDtypeStruct((N, D), jnp.float32),
    )(h, parts, s, w1, b1, g1, be1, w2, b2, g2, be2)


def kernel(x, edge_index, edge_attr, atom_tables, bond_tables,
           W1, b1, W2, b2, eps, g1, be1, g2, be2):
    f32 = jnp.float32
    zrow = jnp.zeros((ZROWS, D), f32)

    tabflat = atom_tables.reshape(ATOM_F * ATOM_V, D).astype(f32)
    xflat = (x.astype(jnp.int32) + jnp.arange(ATOM_F, dtype=jnp.int32)[None, :] * ATOM_V
             ).reshape(-1)
    nid = jnp.repeat(jnp.arange(N, dtype=jnp.int32), ATOM_F)
    xflat = jnp.concatenate([xflat, jnp.zeros((P_PAD - P,), jnp.int32)]
                            ).reshape(P_PAD // 128, 128)
    nid = jnp.concatenate([nid, jnp.full((P_PAD - P,), N, jnp.int32)]
                          ).reshape(P_PAD // 128, 128)

    h = _atom_encode(tabflat, xflat, nid, zrow)

    src = edge_index[0].astype(jnp.int32)
    dst = edge_index[1].astype(jnp.int32)
    ea = edge_attr.astype(jnp.int32)
    code = ea[:, 0] * 25 + ea[:, 1] * 5 + ea[:, 2]
    pad = E_PAD - E
    src = jnp.concatenate([src, jnp.zeros((pad,), jnp.int32)]).reshape(E_PAD // 128, 128)
    dst = jnp.concatenate([dst, jnp.full((pad,), N, jnp.int32)]).reshape(E_PAD // 128, 128)
    code = jnp.concatenate([code, jnp.zeros((pad,), jnp.int32)]).reshape(E_PAD // 128, 128)

    bond = (bond_tables[:, 0][:, :, None, None, :]
            + bond_tables[:, 1][:, None, :, None, :]
            + bond_tables[:, 2][:, None, None, :, :]).reshape(L, 125, D)
    bond = jnp.concatenate([bond, jnp.zeros((L, 3, D), f32)], axis=1)

    for l in range(L):
        parts = _layer_agg(h, src, dst, code, bond[l], zrow)
        s = jnp.reshape(1.0 + eps[l], (1, 1)).astype(f32)
        h = _mlp(h, parts, s,
                 W1[l], b1[l].reshape(1, 2 * D), g1[l].reshape(1, 2 * D),
                 be1[l].reshape(1, 2 * D),
                 W2[l], b2[l].reshape(1, D), g2[l].reshape(1, D),
                 be2[l].reshape(1, D), relu_last=(l < L - 1))
    return h


# speedup vs baseline: 3.3919x; 1.0015x over previous
import functools

import jax
import jax.numpy as jnp
from jax import lax
from jax.experimental import pallas as pl
from jax.experimental.pallas import tpu as pltpu
from jax.experimental.pallas import tpu_sc as plsc

N = 10000
E = 320000
D = 128
L = 3
ATOM_F = 9
ATOM_V = 101

NC = 2
NS = 16
NW = NC * NS

NP = 10240
ZROWS = NP // NS
DROWS = 640

E_ICOPY = 10
E_ROWS_W = E_ICOPY * 8
E_PAD = NW * E_ROWS_W * 128

A_ICOPY = 6
A_ROWS_W = A_ICOPY * 8
P = N * ATOM_F
P_PAD = NS * A_ROWS_W * 128

_MESH = plsc.VectorSubcoreMesh(core_axis_name="c", subcore_axis_name="s")


def _atom_body(tab_hbm, idx_hbm, nid_hbm, zrow_hbm, out_hbm,
               idx_v, nid_v, rows_v, agg_sp, sem):
    cid = lax.axis_index("c")
    sid = lax.axis_index("s")

    @pl.when(cid == 0)
    def _():
        zoff = pl.multiple_of(sid * ZROWS, 8)
        pltpu.sync_copy(zrow_hbm, agg_sp.at[pl.ds(zoff, ZROWS)])
        plsc.subcore_barrier()
        for i in range(A_ICOPY):
            r0 = pl.multiple_of(sid * A_ROWS_W + i * 8, 8)
            pltpu.sync_copy(idx_hbm.at[pl.ds(r0, 8)], idx_v)
            pltpu.sync_copy(nid_hbm.at[pl.ds(r0, 8)], nid_v)
            for j in range(8):
                pltpu.async_copy(tab_hbm.at[idx_v.at[j]], rows_v, sem).wait()
                pltpu.sync_copy(rows_v, agg_sp.at[nid_v.at[j]], add=True)
        plsc.subcore_barrier()

        @pl.when(sid < NS - 1)
        def _():
            doff = pl.multiple_of(sid * DROWS, 8)
            pltpu.sync_copy(agg_sp.at[pl.ds(doff, DROWS)],
                            out_hbm.at[pl.ds(doff, DROWS)])

        @pl.when(sid == NS - 1)
        def _():
            pltpu.sync_copy(agg_sp.at[pl.ds((NS - 1) * DROWS, N - (NS - 1) * DROWS)],
                            out_hbm.at[pl.ds((NS - 1) * DROWS, N - (NS - 1) * DROWS)])


_atom_encode = pl.kernel(
    _atom_body,
    out_type=jax.ShapeDtypeStruct((N, D), jnp.float32),
    mesh=_MESH,
    scratch_types=[
        pltpu.VMEM((8, 128), jnp.int32),
        pltpu.VMEM((8, 128), jnp.int32),
        pltpu.VMEM((128, D), jnp.float32),
        pltpu.VMEM_SHARED((NP, D), jnp.float32),
        pltpu.SemaphoreType.DMA,
    ],
)


def _layer_body(h_hbm, src_hbm, dst_hbm, code_hbm, bond_hbm, zrow_hbm, out_hbm,
                src_v, dst_v, code_v, hr, ee, agg_sp, sem_g, sem_s0, sem_s1):
    cid = lax.axis_index("c")
    sid = lax.axis_index("s")
    w = cid * NS + sid

    zoff = pl.multiple_of(sid * ZROWS, 8)
    pltpu.sync_copy(zrow_hbm, agg_sp.at[pl.ds(zoff, ZROWS)])
    plsc.subcore_barrier()

    def block(i, _):
        r0 = pl.multiple_of(w * E_ROWS_W + i * 8, 8)
        pltpu.sync_copy(src_hbm.at[pl.ds(r0, 8)], src_v)
        pltpu.sync_copy(dst_hbm.at[pl.ds(r0, 8)], dst_v)
        pltpu.sync_copy(code_hbm.at[pl.ds(r0, 8)], code_v)

        def issue_gather(q):
            s = q % 2
            j, half = q // 2, q % 2
            isl = pl.ds(half * 64, 64)
            return (pltpu.async_copy(h_hbm.at[src_v.at[j, isl]], hr.at[s], sem_g),
                    pltpu.async_copy(bond_hbm.at[code_v.at[j, isl]], ee.at[s], sem_g))

        gd = issue_gather(0)
        sc_pend = [None, None]
        for q in range(16):
            s = q % 2
            gd[0].wait()
            gd[1].wait()
            if sc_pend[1 - s] is not None:
                for dsc in sc_pend[1 - s]:
                    dsc.wait()
                sc_pend[1 - s] = None
            if q < 15:
                gd = issue_gather(q + 1)

            def vbody(rr, _):
                for u in range(4):
                    r = rr * 4 + u
                    for c in range(D // 16):
                        sl = pl.ds(c * 16, 16)
                        hr[s, r, sl] = jnp.maximum(hr[s, r, sl] + ee[s, r, sl], 0.0)
                return 0

            lax.fori_loop(0, 16, vbody, 0)

            j, half = q // 2, q % 2
            sem_s = sem_s0 if s == 0 else sem_s1
            scds = []
            for g in range(4):
                dvec = dst_v[j, pl.ds(half * 64 + g * 16, 16)]
                scds.append(pltpu.async_copy(hr.at[s, pl.ds(g * 16, 16)],
                                             agg_sp.at[dvec], sem_s, add=True))
            sc_pend[s] = scds
        for pend in sc_pend:
            if pend is not None:
                for dsc in pend:
                    dsc.wait()
        return 0

    lax.fori_loop(0, E_ICOPY, block, 0)

    plsc.subcore_barrier()

    @pl.when(sid < NS - 1)
    def _():
        doff = pl.multiple_of(sid * DROWS, 8)
        pltpu.sync_copy(agg_sp.at[pl.ds(doff, DROWS)],
                        out_hbm.at[cid, pl.ds(doff, DROWS)])

    @pl.when(sid == NS - 1)
    def _():
        pltpu.sync_copy(agg_sp.at[pl.ds((NS - 1) * DROWS, N - (NS - 1) * DROWS)],
                        out_hbm.at[cid, pl.ds((NS - 1) * DROWS, N - (NS - 1) * DROWS)])


_layer_agg = pl.kernel(
    _layer_body,
    out_type=jax.ShapeDtypeStruct((NC, N, D), jnp.float32),
    mesh=_MESH,
    scratch_types=[
        pltpu.VMEM((8, 128), jnp.int32),
        pltpu.VMEM((8, 128), jnp.int32),
        pltpu.VMEM((8, 128), jnp.int32),
        pltpu.VMEM((2, 64, D), jnp.float32),
        pltpu.VMEM((2, 64, D), jnp.float32),
        pltpu.VMEM_SHARED((NP, D), jnp.float32),
        pltpu.SemaphoreType.DMA,
        pltpu.SemaphoreType.DMA,
        pltpu.SemaphoreType.DMA,
    ],
)


def _mlp_body(relu_last, h_ref, p_ref, s_ref, w1_ref, b1_ref, g1_ref, be1_ref,
              w2_ref, b2_ref, g2_ref, be2_ref, out_ref):
    t = s_ref[0, 0] * h_ref[...] + p_ref[0] + p_ref[1]
    u = jnp.dot(t, w1_ref[...], preferred_element_type=jnp.float32) + b1_ref[...]
    mu = jnp.mean(u, axis=0, keepdims=True)
    var = jnp.mean((u - mu) * (u - mu), axis=0, keepdims=True)
    u = g1_ref[...] * (u - mu) * lax.rsqrt(var + 1e-5) + be1_ref[...]
    u = jnp.maximum(u, 0.0)
    v = jnp.dot(u, w2_ref[...], preferred_element_type=jnp.float32) + b2_ref[...]
    mu2 = jnp.mean(v, axis=0, keepdims=True)
    var2 = jnp.mean((v - mu2) * (v - mu2), axis=0, keepdims=True)
    v = g2_ref[...] * (v - mu2) * lax.rsqrt(var2 + 1e-5) + be2_ref[...]
    if relu_last:
        v = jnp.maximum(v, 0.0)
    out_ref[...] = v


def _mlp(h, parts, s, w1, b1, g1, be1, w2, b2, g2, be2, relu_last):
    return pl.pallas_call(
        functools.partial(_mlp_body, relu_last),
        out_shape=jax.ShapeDtypeStruct((N, D), jnp.float32),
    )(h, parts, s, w1, b1, g1, be1, w2, b2, g2, be2)


def kernel(x, edge_index, edge_attr, atom_tables, bond_tables,
           W1, b1, W2, b2, eps, g1, be1, g2, be2):
    f32 = jnp.float32
    zrow = jnp.zeros((ZROWS, D), f32)

    tabflat = atom_tables.reshape(ATOM_F * ATOM_V, D).astype(f32)
    xflat = (x.astype(jnp.int32) + jnp.arange(ATOM_F, dtype=jnp.int32)[None, :] * ATOM_V
             ).reshape(-1)
    nid = jnp.repeat(jnp.arange(N, dtype=jnp.int32), ATOM_F)
    xflat = jnp.concatenate([xflat, jnp.zeros((P_PAD - P,), jnp.int32)]
                            ).reshape(P_PAD // 128, 128)
    nid = jnp.concatenate([nid, jnp.full((P_PAD - P,), N, jnp.int32)]
                          ).reshape(P_PAD // 128, 128)

    h = _atom_encode(tabflat, xflat, nid, zrow)

    src = edge_index[0].astype(jnp.int32)
    dst = edge_index[1].astype(jnp.int32)
    ea = edge_attr.astype(jnp.int32)
    code = ea[:, 0] * 25 + ea[:, 1] * 5 + ea[:, 2]
    pad = E_PAD - E
    src = jnp.concatenate([src, jnp.zeros((pad,), jnp.int32)]).reshape(E_PAD // 128, 128)
    dst = jnp.concatenate([dst, jnp.full((pad,), N, jnp.int32)]).reshape(E_PAD // 128, 128)
    code = jnp.concatenate([code, jnp.zeros((pad,), jnp.int32)]).reshape(E_PAD // 128, 128)

    bond = (bond_tables[:, 0][:, :, None, None, :]
            + bond_tables[:, 1][:, None, :, None, :]
            + bond_tables[:, 2][:, None, None, :, :]).reshape(L, 125, D)
    bond = jnp.concatenate([bond, jnp.zeros((L, 3, D), f32)], axis=1)

    for l in range(L):
        parts = _layer_agg(h, src, dst, code, bond[l], zrow)
        s = jnp.reshape(1.0 + eps[l], (1, 1)).astype(f32)
        h = _mlp(h, parts, s,
                 W1[l], b1[l].reshape(1, 2 * D), g1[l].reshape(1, 2 * D),
                 be1[l].reshape(1, 2 * D),
                 W2[l], b2[l].reshape(1, D), g2[l].reshape(1, D),
                 be2[l].reshape(1, D), relu_last=(l < L - 1))
    return h


# speedup vs baseline: 3.7094x; 1.0936x over previous
import functools

import jax
import jax.numpy as jnp
from jax import lax
from jax.experimental import pallas as pl
from jax.experimental.pallas import tpu as pltpu
from jax.experimental.pallas import tpu_sc as plsc

N = 10000
E = 320000
D = 128
L = 3
ATOM_F = 9
ATOM_V = 101

NC = 2
NS = 16
NW = NC * NS

NP = 10240
ZROWS = NP // NS
DROWS = 640

E_ICOPY = 10
E_ROWS_W = E_ICOPY * 8
E_PAD = NW * E_ROWS_W * 128

A_ICOPY = 3
A_ROWS_W = A_ICOPY * 8
P = N * ATOM_F
P_HALF = (N // 2) * ATOM_F
P_PAD_HALF = NS * A_ROWS_W * 128
P_PAD = NC * P_PAD_HALF
A_DROWS = 320

_MESH = plsc.VectorSubcoreMesh(core_axis_name="c", subcore_axis_name="s")


def _atom_body(tab_hbm, idx_hbm, nid_hbm, zrow_hbm, out_hbm,
               idx_v, nid_v, rows_v, agg_sp, sem_g, sem_s0, sem_s1):
    cid = lax.axis_index("c")
    sid = lax.axis_index("s")

    zoff = pl.multiple_of(sid * ZROWS, 8)
    pltpu.sync_copy(zrow_hbm, agg_sp.at[pl.ds(zoff, ZROWS)])
    plsc.subcore_barrier()

    r0 = pl.multiple_of(cid * (NS * A_ROWS_W) + sid * A_ROWS_W, 8)
    pltpu.sync_copy(idx_hbm.at[pl.ds(r0, A_ROWS_W)], idx_v)
    pltpu.sync_copy(nid_hbm.at[pl.ds(r0, A_ROWS_W)], nid_v)

    gd = pltpu.async_copy(tab_hbm.at[idx_v.at[0]], rows_v.at[0], sem_g)
    sc_pend = [None, None]
    for j in range(A_ROWS_W):
        s = j % 2
        gd.wait()
        if sc_pend[1 - s] is not None:
            sc_pend[1 - s].wait()
            sc_pend[1 - s] = None
        if j < A_ROWS_W - 1:
            gd = pltpu.async_copy(tab_hbm.at[idx_v.at[j + 1]],
                                  rows_v.at[1 - s], sem_g)
        sem_s = sem_s0 if s == 0 else sem_s1
        sc_pend[s] = pltpu.async_copy(rows_v.at[s], agg_sp.at[nid_v.at[j]],
                                      sem_s, add=True)
    for pend in sc_pend:
        if pend is not None:
            pend.wait()

    plsc.subcore_barrier()
    base = cid * (N // 2)

    @pl.when(sid < NS - 1)
    def _():
        doff = pl.multiple_of(base + sid * A_DROWS, 8)
        pltpu.sync_copy(agg_sp.at[pl.ds(doff, A_DROWS)],
                        out_hbm.at[pl.ds(doff, A_DROWS)])

    @pl.when(sid == NS - 1)
    def _():
        tail = N // 2 - (NS - 1) * A_DROWS
        doff = pl.multiple_of(base + (NS - 1) * A_DROWS, 8)
        pltpu.sync_copy(agg_sp.at[pl.ds(doff, tail)],
                        out_hbm.at[pl.ds(doff, tail)])


_atom_encode = pl.kernel(
    _atom_body,
    out_type=jax.ShapeDtypeStruct((N, D), jnp.float32),
    mesh=_MESH,
    scratch_types=[
        pltpu.VMEM((A_ROWS_W, 128), jnp.int32),
        pltpu.VMEM((A_ROWS_W, 128), jnp.int32),
        pltpu.VMEM((2, 128, D), jnp.float32),
        pltpu.VMEM_SHARED((NP, D), jnp.float32),
        pltpu.SemaphoreType.DMA,
        pltpu.SemaphoreType.DMA,
        pltpu.SemaphoreType.DMA,
    ],
)


def _layer_body(h_hbm, src_hbm, dst_hbm, code_hbm, bond_hbm, zrow_hbm, out_hbm,
                src_v, dst_v, code_v, hr, ee, agg_sp, sem_g, sem_s0, sem_s1):
    cid = lax.axis_index("c")
    sid = lax.axis_index("s")
    w = cid * NS + sid

    zoff = pl.multiple_of(sid * ZROWS, 8)
    pltpu.sync_copy(zrow_hbm, agg_sp.at[pl.ds(zoff, ZROWS)])
    plsc.subcore_barrier()

    def block(i, _):
        r0 = pl.multiple_of(w * E_ROWS_W + i * 8, 8)
        pltpu.sync_copy(src_hbm.at[pl.ds(r0, 8)], src_v)
        pltpu.sync_copy(dst_hbm.at[pl.ds(r0, 8)], dst_v)
        pltpu.sync_copy(code_hbm.at[pl.ds(r0, 8)], code_v)

        def issue_gather(q):
            s = q % 2
            j, half = q // 2, q % 2
            isl = pl.ds(half * 64, 64)
            return (pltpu.async_copy(h_hbm.at[src_v.at[j, isl]], hr.at[s], sem_g),
                    pltpu.async_copy(bond_hbm.at[code_v.at[j, isl]], ee.at[s], sem_g))

        gd = issue_gather(0)
        sc_pend = [None, None]
        for q in range(16):
            s = q % 2
            gd[0].wait()
            gd[1].wait()
            if sc_pend[1 - s] is not None:
                for dsc in sc_pend[1 - s]:
                    dsc.wait()
                sc_pend[1 - s] = None
            if q < 15:
                gd = issue_gather(q + 1)

            def vbody(rr, _):
                for u in range(4):
                    r = rr * 4 + u
                    for c in range(D // 16):
                        sl = pl.ds(c * 16, 16)
                        hr[s, r, sl] = jnp.maximum(hr[s, r, sl] + ee[s, r, sl], 0.0)
                return 0

            lax.fori_loop(0, 16, vbody, 0)

            j, half = q // 2, q % 2
            sem_s = sem_s0 if s == 0 else sem_s1
            scds = []
            for g in range(4):
                dvec = dst_v[j, pl.ds(half * 64 + g * 16, 16)]
                scds.append(pltpu.async_copy(hr.at[s, pl.ds(g * 16, 16)],
                                             agg_sp.at[dvec], sem_s, add=True))
            sc_pend[s] = scds
        for pend in sc_pend:
            if pend is not None:
                for dsc in pend:
                    dsc.wait()
        return 0

    lax.fori_loop(0, E_ICOPY, block, 0)

    plsc.subcore_barrier()

    @pl.when(sid < NS - 1)
    def _():
        doff = pl.multiple_of(sid * DROWS, 8)
        pltpu.sync_copy(agg_sp.at[pl.ds(doff, DROWS)],
                        out_hbm.at[cid, pl.ds(doff, DROWS)])

    @pl.when(sid == NS - 1)
    def _():
        pltpu.sync_copy(agg_sp.at[pl.ds((NS - 1) * DROWS, N - (NS - 1) * DROWS)],
                        out_hbm.at[cid, pl.ds((NS - 1) * DROWS, N - (NS - 1) * DROWS)])


_layer_agg = pl.kernel(
    _layer_body,
    out_type=jax.ShapeDtypeStruct((NC, N, D), jnp.float32),
    mesh=_MESH,
    scratch_types=[
        pltpu.VMEM((8, 128), jnp.int32),
        pltpu.VMEM((8, 128), jnp.int32),
        pltpu.VMEM((8, 128), jnp.int32),
        pltpu.VMEM((2, 64, D), jnp.float32),
        pltpu.VMEM((2, 64, D), jnp.float32),
        pltpu.VMEM_SHARED((NP, D), jnp.float32),
        pltpu.SemaphoreType.DMA,
        pltpu.SemaphoreType.DMA,
        pltpu.SemaphoreType.DMA,
    ],
)


def _mlp_body(relu_last, h_ref, p_ref, s_ref, w1_ref, b1_ref, g1_ref, be1_ref,
              w2_ref, b2_ref, g2_ref, be2_ref, out_ref):
    t = s_ref[0, 0] * h_ref[...] + p_ref[0] + p_ref[1]
    u = jnp.dot(t, w1_ref[...], preferred_element_type=jnp.float32) + b1_ref[...]
    mu = jnp.mean(u, axis=0, keepdims=True)
    var = jnp.mean((u - mu) * (u - mu), axis=0, keepdims=True)
    u = g1_ref[...] * (u - mu) * lax.rsqrt(var + 1e-5) + be1_ref[...]
    u = jnp.maximum(u, 0.0)
    v = jnp.dot(u, w2_ref[...], preferred_element_type=jnp.float32) + b2_ref[...]
    mu2 = jnp.mean(v, axis=0, keepdims=True)
    var2 = jnp.mean((v - mu2) * (v - mu2), axis=0, keepdims=True)
    v = g2_ref[...] * (v - mu2) * lax.rsqrt(var2 + 1e-5) + be2_ref[...]
    if relu_last:
        v = jnp.maximum(v, 0.0)
    out_ref[...] = v


def _mlp(h, parts, s, w1, b1, g1, be1, w2, b2, g2, be2, relu_last):
    return pl.pallas_call(
        functools.partial(_mlp_body, relu_last),
        out_shape=jax.ShapeDtypeStruct((N, D), jnp.float32),
    )(h, parts, s, w1, b1, g1, be1, w2, b2, g2, be2)


def kernel(x, edge_index, edge_attr, atom_tables, bond_tables,
           W1, b1, W2, b2, eps, g1, be1, g2, be2):
    f32 = jnp.float32
    zrow = jnp.zeros((ZROWS, D), f32)

    tabflat = atom_tables.reshape(ATOM_F * ATOM_V, D).astype(f32)
    xflat = (x.astype(jnp.int32) + jnp.arange(ATOM_F, dtype=jnp.int32)[None, :] * ATOM_V
             ).reshape(-1)
    nid = jnp.repeat(jnp.arange(N, dtype=jnp.int32), ATOM_F)
    padh = P_PAD_HALF - P_HALF
    xflat = jnp.concatenate([
        xflat[:P_HALF], jnp.zeros((padh,), jnp.int32),
        xflat[P_HALF:], jnp.zeros((padh,), jnp.int32),
    ]).reshape(P_PAD // 128, 128)
    nid = jnp.concatenate([
        nid[:P_HALF], jnp.full((padh,), N, jnp.int32),
        nid[P_HALF:], jnp.full((padh,), N, jnp.int32),
    ]).reshape(P_PAD // 128, 128)

    h = _atom_encode(tabflat, xflat, nid, zrow)

    src = edge_index[0].astype(jnp.int32)
    dst = edge_index[1].astype(jnp.int32)
    ea = edge_attr.astype(jnp.int32)
    code = ea[:, 0] * 25 + ea[:, 1] * 5 + ea[:, 2]
    pad = E_PAD - E
    src = jnp.concatenate([src, jnp.zeros((pad,), jnp.int32)]).reshape(E_PAD // 128, 128)
    dst = jnp.concatenate([dst, jnp.full((pad,), N, jnp.int32)]).reshape(E_PAD // 128, 128)
    code = jnp.concatenate([code, jnp.zeros((pad,), jnp.int32)]).reshape(E_PAD // 128, 128)

    bond = (bond_tables[:, 0][:, :, None, None, :]
            + bond_tables[:, 1][:, None, :, None, :]
            + bond_tables[:, 2][:, None, None, :, :]).reshape(L, 125, D)
    bond = jnp.concatenate([bond, jnp.zeros((L, 3, D), f32)], axis=1)

    for l in range(L):
        parts = _layer_agg(h, src, dst, code, bond[l], zrow)
        s = jnp.reshape(1.0 + eps[l], (1, 1)).astype(f32)
        h = _mlp(h, parts, s,
                 W1[l], b1[l].reshape(1, 2 * D), g1[l].reshape(1, 2 * D),
                 be1[l].reshape(1, 2 * D),
                 W2[l], b2[l].reshape(1, D), g2[l].reshape(1, D),
                 be2[l].reshape(1, D), relu_last=(l < L - 1))
    return h
